# Initial kernel scaffold; baseline (speedup 1.0000x reference)
import jax
import jax.numpy as jnp
from jax.experimental import pallas as pl


def kernel(x1, x2, x3, x4, w_fc_t, w_fc1_t, w_fc2_t, w_fc3_t, w_fc4_t, w_m1_t, w_m2_t):
    raise NotImplementedError("write your pallas kernel here")


# speedup vs baseline: 1.2350x; 1.2350x over previous
import jax
import jax.numpy as jnp
from jax.experimental import pallas as pl
from jax.experimental.pallas import tpu as pltpu


def _silu(x):
    return x * jax.nn.sigmoid(x)


def _softmax_lanes(v):
    v = v - jnp.max(v, axis=1, keepdims=True)
    e = jnp.exp(v)
    return e / jnp.sum(e, axis=1, keepdims=True)


def _fused_kernel(x1_ref, x2_ref, x3_ref, x4_ref,
                  wfc_ref, w1_ref, w2_ref, w3_ref, w4_ref,
                  wm1_ref, wm2_ref, o_ref):
    f32 = jnp.float32
    hw = x1_ref.shape[2]
    inv_hw = f32(1.0 / hw)

    x1 = x1_ref[...]
    x2 = x2_ref[...]
    x3 = x3_ref[...]
    x4 = x4_ref[...]

    m1 = jnp.sum(x1, axis=2) * inv_hw
    m2 = jnp.sum(x2, axis=2) * inv_hw
    m3 = jnp.sum(x3, axis=2) * inv_hw
    m4 = jnp.sum(x4, axis=2) * inv_hw
    m_sum = m1 + m2 + m3 + m4

    y = _silu(jnp.dot(m_sum, wfc_ref[...], preferred_element_type=f32))
    z1 = _softmax_lanes(_silu(jnp.dot(y, w1_ref[...], preferred_element_type=f32)))
    z2 = _softmax_lanes(_silu(jnp.dot(y, w2_ref[...], preferred_element_type=f32)))
    z3 = _softmax_lanes(_silu(jnp.dot(y, w3_ref[...], preferred_element_type=f32)))
    z4 = _softmax_lanes(_silu(jnp.dot(y, w4_ref[...], preferred_element_type=f32)))

    p1 = m1 * z1
    p2 = m2 * z2
    p3 = m3 * z3
    p4 = m4 * z4
    h = (jnp.dot(p1, wm1_ref[0], preferred_element_type=f32)
         + jnp.dot(p2, wm1_ref[1], preferred_element_type=f32)
         + jnp.dot(p3, wm1_ref[2], preferred_element_type=f32)
         + jnp.dot(p4, wm1_ref[3], preferred_element_type=f32))
    h = _silu(h)
    a = _silu(jnp.dot(h, wm2_ref[...], preferred_element_type=f32))

    g1 = a[:, 0:1] * z1
    g2 = a[:, 1:2] * z2
    g3 = a[:, 2:3] * z3
    g4 = a[:, 3:4] * z4

    out = g1[:, :, None] * x1
    out += g2[:, :, None] * x2
    out += g3[:, :, None] * x3
    out += g4[:, :, None] * x4
    o_ref[...] = out.astype(o_ref.dtype)


def kernel(x1, x2, x3, x4, w_fc_t, w_fc1_t, w_fc2_t, w_fc3_t, w_fc4_t,
           w_m1_t, w_m2_t):
    B, C, H, W = x1.shape
    HW = H * W
    xs = [x.reshape(B, C, HW) for x in (x1, x2, x3, x4)]
    hid = w_fc_t.shape[1]
    hid4 = w_m1_t.shape[1]
    wm1 = w_m1_t.reshape(4, C, hid4)

    x_spec = pl.BlockSpec((1, C, HW), lambda b: (b, 0, 0))
    wfc_spec = pl.BlockSpec((C, hid), lambda b: (0, 0))
    wx_spec = pl.BlockSpec((hid, C), lambda b: (0, 0))
    wm1_spec = pl.BlockSpec((4, C, hid4), lambda b: (0, 0, 0))
    wm2_spec = pl.BlockSpec((hid4, 4), lambda b: (0, 0))

    out = pl.pallas_call(
        _fused_kernel,
        out_shape=jax.ShapeDtypeStruct((B, C, HW), x1.dtype),
        grid=(B,),
        in_specs=[x_spec, x_spec, x_spec, x_spec,
                  wfc_spec, wx_spec, wx_spec, wx_spec, wx_spec,
                  wm1_spec, wm2_spec],
        out_specs=x_spec,
        compiler_params=pltpu.CompilerParams(
            dimension_semantics=("parallel",),
            vmem_limit_bytes=60 * 1024 * 1024),
    )(*xs, w_fc_t, w_fc1_t, w_fc2_t, w_fc3_t, w_fc4_t, wm1, w_m2_t)
    return out.reshape(B, C, H, W)
